# baseline (speedup 1.0000x reference)
import jax
import jax.numpy as jnp
from jax import lax
from jax.experimental import pallas as pl
from jax.experimental.pallas import tpu as pltpu
from jax.experimental.pallas import tpu_sc as plsc

NUM_NODES = 100000
MEM_DIM = 128
BATCH = 16384

NC = 2
NS = 16
NW = NC * NS
LANES = 16

BPW = BATCH // NW
GCH = 128
RANGE = 3128
LAST_RANGE = NUM_NODES - (NW - 1) * RANGE
UCH = 128
CCH = 256
RPAD = ((RANGE + CCH - 1) // CCH) * CCH


def _widx():
  return lax.axis_index("s") * NC + lax.axis_index("c")


def _mesh():
  return plsc.VectorSubcoreMesh(core_axis_name="c", subcore_axis_name="s")


def _gather_scan_body(ids_hbm, mem_hbm, h_hbm, w_hbm,
                      ids2d, ids_all, wtab, rows_v, sem_g):
  wid = _widx()
  base = wid * BPW
  nch = BPW // GCH

  for k in range(nch):
    pltpu.sync_copy(ids_hbm.at[pl.ds(base + k * GCH, GCH)], ids2d.at[k])
  for k in range(nch):
    pltpu.async_copy(mem_hbm.at[ids2d.at[k]],
                     rows_v.at[pl.ds(k * GCH, GCH)], sem_g)

  lo = wid * RANGE
  pltpu.sync_copy(ids_hbm, ids_all)
  neg1 = jnp.full((LANES,), -1, jnp.int32)
  def _init(j, _):
    wtab[pl.ds(j * LANES, LANES)] = neg1
    return 0
  lax.fori_loop(0, RPAD // LANES, _init, 0, unroll=8)

  iota = lax.broadcasted_iota(jnp.int32, (LANES,), 0)
  def _scan(i, _):
    v = ids_all[pl.ds(i * LANES, LANES)]
    pos = iota + i * LANES
    _, win = plsc.scan_count(v)
    inr = jnp.logical_and(v >= lo, v < lo + RANGE)
    plsc.store_scatter(wtab, [v - lo], pos, mask=jnp.logical_and(win, inr))
    return 0
  lax.fori_loop(0, BATCH // LANES, _scan, 0, unroll=8)
  pltpu.sync_copy(wtab, w_hbm.at[pl.ds(wid * RPAD, RPAD)])

  for k in range(nch):
    pltpu.make_async_copy(mem_hbm.at[ids2d.at[k]],
                          rows_v.at[pl.ds(k * GCH, GCH)], sem_g).wait()
  pltpu.sync_copy(rows_v, h_hbm.at[pl.ds(base, BPW)])


def _sc_gather_scan(node_ids, memory):
  f = pl.kernel(
      _gather_scan_body,
      out_type=(
          jax.ShapeDtypeStruct((BATCH, MEM_DIM), jnp.float32),
          jax.ShapeDtypeStruct((NW * RPAD,), jnp.int32),
      ),
      mesh=_mesh(),
      scratch_types=[
          pltpu.VMEM((BPW // GCH, GCH), jnp.int32),
          pltpu.VMEM((BATCH,), jnp.int32),
          pltpu.VMEM((RPAD,), jnp.int32),
          pltpu.VMEM((BPW, MEM_DIM), jnp.float32),
          pltpu.SemaphoreType.DMA,
      ],
      compiler_params=pltpu.CompilerParams(needs_layout_passes=False),
  )
  return f(node_ids, memory)


def _gru_body(m_ref, h_ref, wih_ref, whh_ref, bih_ref, bhh_ref, out_ref):
  m = m_ref[...]
  h = h_ref[...]
  dn = (((1,), (1,)), ((), ()))
  gi = lax.dot_general(m, wih_ref[...], dn,
                       preferred_element_type=jnp.float32)
  gh = lax.dot_general(h, whh_ref[...], dn,
                       preferred_element_type=jnp.float32)
  gi = gi + bih_ref[0:1, :]
  gh = gh + bhh_ref[0:1, :]
  i_r = gi[:, 0:MEM_DIM]
  i_z = gi[:, MEM_DIM:2 * MEM_DIM]
  i_n = gi[:, 2 * MEM_DIM:3 * MEM_DIM]
  h_r = gh[:, 0:MEM_DIM]
  h_z = gh[:, MEM_DIM:2 * MEM_DIM]
  h_n = gh[:, 2 * MEM_DIM:3 * MEM_DIM]
  r = jax.nn.sigmoid(i_r + h_r)
  z = jax.nn.sigmoid(i_z + h_z)
  n = jnp.tanh(i_n + r * h_n)
  out_ref[...] = (1.0 - z) * n + z * h


def _tc_gru(messages, h, W_ih, W_hh, b_ih, b_hh):
  bm = 2048
  grid = (BATCH // bm,)
  return pl.pallas_call(
      _gru_body,
      grid=grid,
      in_specs=[
          pl.BlockSpec((bm, MEM_DIM), lambda i: (i, 0)),
          pl.BlockSpec((bm, MEM_DIM), lambda i: (i, 0)),
          pl.BlockSpec((3 * MEM_DIM, MEM_DIM), lambda i: (0, 0)),
          pl.BlockSpec((3 * MEM_DIM, MEM_DIM), lambda i: (0, 0)),
          pl.BlockSpec((8, 3 * MEM_DIM), lambda i: (0, 0)),
          pl.BlockSpec((8, 3 * MEM_DIM), lambda i: (0, 0)),
      ],
      out_specs=pl.BlockSpec((bm, MEM_DIM), lambda i: (i, 0)),
      out_shape=jax.ShapeDtypeStruct((BATCH, MEM_DIM), jnp.float32),
      compiler_params=pltpu.CompilerParams(
          dimension_semantics=("arbitrary",)),
  )(messages, h, W_ih, W_hh, b_ih, b_hh)


TCB = 2000

def _tc_copy_body(mem_ref, out_ref):
  out_ref[...] = mem_ref[...]


def _tc_copy(memory):
  return pl.pallas_call(
      _tc_copy_body,
      grid=(NUM_NODES // TCB,),
      in_specs=[pl.BlockSpec((TCB, MEM_DIM), lambda i: (i, 0))],
      out_specs=pl.BlockSpec((TCB, MEM_DIM), lambda i: (i, 0)),
      out_shape=jax.ShapeDtypeStruct((NUM_NODES, MEM_DIM), jnp.float32),
      compiler_params=pltpu.CompilerParams(
          dimension_semantics=("arbitrary",)),
  )(memory)


def _update_body(w_hbm, newh_hbm, out_ref,
                 wtab, idx_stage, rows0, rows1, sem_u0, sem_u1):
  wid = _widx()
  lo = wid * RANGE
  pltpu.sync_copy(w_hbm.at[pl.ds(wid * RPAD, RPAD)], wtab)

  iota = lax.broadcasted_iota(jnp.int32, (LANES,), 0)
  rows = (rows0, rows1)
  sems = (sem_u0, sem_u1)
  nch = RPAD // UCH

  def _gather_chunk(c):
    b = c & 1
    src = newh_hbm.at[plsc.Indices(wtab.at[pl.ds(c * UCH, UCH)],
                                   ignored_value=-1)]
    return pltpu.async_copy(src, rows[b], sems[b])

  def _scatter_chunk(c, d):
    b = c & 1
    d.wait()
    base = c * UCH
    for v in range(UCH // LANES):
      wv = wtab[pl.ds(base + v * LANES, LANES)]
      row = iota + (lo + base + v * LANES)
      idx_stage[pl.ds(v * LANES, LANES)] = jnp.where(wv >= 0, row, -1)
    dst = out_ref.at[plsc.Indices(idx_stage, ignored_value=-1)]
    pltpu.sync_copy(rows[b], dst)

  prev = _gather_chunk(0)
  for c in range(1, nch):
    cur = _gather_chunk(c)
    _scatter_chunk(c - 1, prev)
    prev = cur
  _scatter_chunk(nch - 1, prev)


def _sc_update(wvec, new_h, out_ref):
  f = pl.kernel(
      _update_body,
      out_type=(),
      mesh=_mesh(),
      scratch_types=[
          pltpu.VMEM((RPAD,), jnp.int32),
          pltpu.VMEM((UCH,), jnp.int32),
          pltpu.VMEM((UCH, MEM_DIM), jnp.float32),
          pltpu.VMEM((UCH, MEM_DIM), jnp.float32),
          pltpu.SemaphoreType.DMA,
          pltpu.SemaphoreType.DMA,
      ],
      compiler_params=pltpu.CompilerParams(needs_layout_passes=False),
  )
  f(wvec, new_h, out_ref)


def _merge_body(mem_hbm, w_hbm, newh_hbm, out_hbm,
                wtab, cbuf0, cbuf1, sem_c0, sem_c1, sem_o0, sem_o1):
  wid = _widx()
  lo = wid * RANGE
  is_last = wid == NW - 1
  pltpu.sync_copy(w_hbm.at[pl.ds(wid * RPAD, RPAD)], wtab)

  bufs = (cbuf0, cbuf1)
  csems = (sem_c0, sem_c1)
  osems = (sem_o0, sem_o1)

  def _run(n_rows):
    sizes = [CCH] * (n_rows // CCH)
    if n_rows % CCH:
      sizes.append(n_rows % CCH)
    offs = [sum(sizes[:k]) for k in range(len(sizes))]

    def _issue_in(k):
      b = k & 1
      return pltpu.async_copy(mem_hbm.at[pl.ds(lo + offs[k], sizes[k])],
                              bufs[b].at[pl.ds(0, sizes[k])], csems[b])

    d_in = _issue_in(0)
    for k, (off, sz) in enumerate(zip(offs, sizes)):
      b = k & 1
      d_in.wait()
      ovs = []
      for so in range(0, sz, UCH):
        ss = min(UCH, sz - so)
        src = newh_hbm.at[plsc.Indices(wtab.at[pl.ds(off + so, ss)],
                                       ignored_value=-1)]
        ovs.append(pltpu.async_copy(src, bufs[b].at[pl.ds(so, ss)],
                                    osems[b]))
      if k + 1 < len(sizes):
        d_in = _issue_in(k + 1)
      for d in ovs:
        d.wait()
      pltpu.sync_copy(bufs[b].at[pl.ds(0, sz)],
                      out_hbm.at[pl.ds(lo + off, sz)])

  @pl.when(jnp.logical_not(is_last))
  def _():
    _run(RANGE)
  @pl.when(is_last)
  def _():
    _run(LAST_RANGE)


def _sc_merge(memory, wvec, new_h):
  f = pl.kernel(
      _merge_body,
      out_type=jax.ShapeDtypeStruct((NUM_NODES, MEM_DIM), jnp.float32),
      mesh=_mesh(),
      scratch_types=[
          pltpu.VMEM((RPAD,), jnp.int32),
          pltpu.VMEM((CCH, MEM_DIM), jnp.float32),
          pltpu.VMEM((CCH, MEM_DIM), jnp.float32),
          pltpu.SemaphoreType.DMA,
          pltpu.SemaphoreType.DMA,
          pltpu.SemaphoreType.DMA,
          pltpu.SemaphoreType.DMA,
      ],
      compiler_params=pltpu.CompilerParams(needs_layout_passes=False),
  )
  return f(memory, wvec, new_h)


def kernel(node_ids, messages, memory, W_ih, W_hh, b_ih, b_hh):
  node_ids = node_ids.astype(jnp.int32)
  b_ih2 = jnp.broadcast_to(b_ih.reshape(1, -1), (8, 3 * MEM_DIM))
  b_hh2 = jnp.broadcast_to(b_hh.reshape(1, -1), (8, 3 * MEM_DIM))
  h, wvec = _sc_gather_scan(node_ids, memory)
  out0 = _tc_copy(memory)
  new_h = _tc_gru(messages, h, W_ih, W_hh, b_ih2, b_hh2)
  out_ref = jax.new_ref(out0)
  _sc_update(wvec, new_h, out_ref)
  return out_ref[...]


# speedup vs baseline: 1.0161x; 1.0161x over previous
import jax
import jax.numpy as jnp
from jax import lax
from jax.experimental import pallas as pl
from jax.experimental.pallas import tpu as pltpu
from jax.experimental.pallas import tpu_sc as plsc

NUM_NODES = 100000
MEM_DIM = 128
BATCH = 16384

NC = 2
NS = 16
NW = NC * NS
LANES = 16

BPW = BATCH // NW
GCH = 128
RANGE = 3128
LAST_RANGE = NUM_NODES - (NW - 1) * RANGE
UCH = 128
CCH = 256
RPAD = ((RANGE + CCH - 1) // CCH) * CCH
NSCAN = BATCH // LANES


def _widx():
  return lax.axis_index("s") * NC + lax.axis_index("c")


def _mesh():
  return plsc.VectorSubcoreMesh(core_axis_name="c", subcore_axis_name="s")


def _gather_body(ids_hbm, mem_hbm, h_hbm, ids2d, rows_v, sem_g):
  wid = _widx()
  base = wid * BPW
  nch = BPW // GCH
  for k in range(nch):
    pltpu.sync_copy(ids_hbm.at[pl.ds(base + k * GCH, GCH)], ids2d.at[k])
  for k in range(nch):
    pltpu.async_copy(mem_hbm.at[ids2d.at[k]],
                     rows_v.at[pl.ds(k * GCH, GCH)], sem_g)
  for k in range(nch):
    pltpu.make_async_copy(mem_hbm.at[ids2d.at[k]],
                          rows_v.at[pl.ds(k * GCH, GCH)], sem_g).wait()
  pltpu.sync_copy(rows_v, h_hbm.at[pl.ds(base, BPW)])


def _sc_gather(node_ids, memory):
  f = pl.kernel(
      _gather_body,
      out_type=jax.ShapeDtypeStruct((BATCH, MEM_DIM), jnp.float32),
      mesh=_mesh(),
      scratch_types=[
          pltpu.VMEM((BPW // GCH, GCH), jnp.int32),
          pltpu.VMEM((BPW, MEM_DIM), jnp.float32),
          pltpu.SemaphoreType.DMA,
      ],
      compiler_params=pltpu.CompilerParams(needs_layout_passes=False),
  )
  return f(node_ids, memory)


def _gru_body(m_ref, h_ref, wih_ref, whh_ref, bih_ref, bhh_ref, out_ref):
  m = m_ref[...]
  h = h_ref[...]
  dn = (((1,), (1,)), ((), ()))
  gi = lax.dot_general(m, wih_ref[...], dn,
                       preferred_element_type=jnp.float32)
  gh = lax.dot_general(h, whh_ref[...], dn,
                       preferred_element_type=jnp.float32)
  gi = gi + bih_ref[0:1, :]
  gh = gh + bhh_ref[0:1, :]
  i_r = gi[:, 0:MEM_DIM]
  i_z = gi[:, MEM_DIM:2 * MEM_DIM]
  i_n = gi[:, 2 * MEM_DIM:3 * MEM_DIM]
  h_r = gh[:, 0:MEM_DIM]
  h_z = gh[:, MEM_DIM:2 * MEM_DIM]
  h_n = gh[:, 2 * MEM_DIM:3 * MEM_DIM]
  r = jax.nn.sigmoid(i_r + h_r)
  z = jax.nn.sigmoid(i_z + h_z)
  n = jnp.tanh(i_n + r * h_n)
  out_ref[...] = (1.0 - z) * n + z * h


def _tc_gru(messages, h, W_ih, W_hh, b_ih, b_hh):
  bm = 2048
  grid = (BATCH // bm,)
  return pl.pallas_call(
      _gru_body,
      grid=grid,
      in_specs=[
          pl.BlockSpec((bm, MEM_DIM), lambda i: (i, 0)),
          pl.BlockSpec((bm, MEM_DIM), lambda i: (i, 0)),
          pl.BlockSpec((3 * MEM_DIM, MEM_DIM), lambda i: (0, 0)),
          pl.BlockSpec((3 * MEM_DIM, MEM_DIM), lambda i: (0, 0)),
          pl.BlockSpec((8, 3 * MEM_DIM), lambda i: (0, 0)),
          pl.BlockSpec((8, 3 * MEM_DIM), lambda i: (0, 0)),
      ],
      out_specs=pl.BlockSpec((bm, MEM_DIM), lambda i: (i, 0)),
      out_shape=jax.ShapeDtypeStruct((BATCH, MEM_DIM), jnp.float32),
      compiler_params=pltpu.CompilerParams(
          dimension_semantics=("arbitrary",)),
  )(messages, h, W_ih, W_hh, b_ih, b_hh)


def _merge_body(ids_hbm, mem_hbm, newh_hbm, out_hbm,
                ids_all, wtab, idx_stage, cbuf0, cbuf1,
                sem_c0, sem_c1, sem_u0, sem_u1):
  wid = _widx()
  lo = wid * RANGE
  is_last = wid == NW - 1
  iota = lax.broadcasted_iota(jnp.int32, (LANES,), 0)
  bufs = (cbuf0, cbuf1)
  csems = (sem_c0, sem_c1)
  usems = (sem_u0, sem_u1)

  d_ids = pltpu.async_copy(ids_hbm, ids_all, sem_u0)
  neg1 = jnp.full((LANES,), -1, jnp.int32)
  def _init(j, _):
    wtab[pl.ds(j * LANES, LANES)] = neg1
    return 0
  lax.fori_loop(0, RPAD // LANES, _init, 0, unroll=8)
  d_ids.wait()

  def _scan(i, _):
    v = ids_all[pl.ds(i * LANES, LANES)]
    pos = iota + i * LANES
    _, win = plsc.scan_count(v)
    inr = jnp.logical_and(v >= lo, v < lo + RANGE)
    plsc.store_scatter(wtab, [v - lo], pos, mask=jnp.logical_and(win, inr))
    return 0

  def _run(n_rows):
    sizes = [CCH] * (n_rows // CCH)
    if n_rows % CCH:
      sizes.append(n_rows % CCH)
    offs = [sum(sizes[:k]) for k in range(len(sizes))]
    nchunks = len(sizes)
    bounds = [(k * NSCAN) // nchunks for k in range(nchunks)] + [NSCAN]

    def _issue_in(k):
      b = k & 1
      return pltpu.async_copy(mem_hbm.at[pl.ds(lo + offs[k], sizes[k])],
                              bufs[b].at[pl.ds(0, sizes[k])], csems[b])

    d_in = _issue_in(0)
    for k in range(nchunks):
      b = k & 1
      d_nxt = _issue_in(k + 1) if k + 1 < nchunks else None
      lax.fori_loop(bounds[k], bounds[k + 1], _scan, 0, unroll=4)
      d_in.wait()
      pltpu.sync_copy(bufs[b].at[pl.ds(0, sizes[k])],
                      out_hbm.at[pl.ds(lo + offs[k], sizes[k])])
      d_in = d_nxt

  @pl.when(jnp.logical_not(is_last))
  def _():
    _run(RANGE)
  @pl.when(is_last)
  def _():
    _run(LAST_RANGE)

  nup = RPAD // UCH

  def _gather_chunk(c):
    b = c & 1
    src = newh_hbm.at[plsc.Indices(wtab.at[pl.ds(c * UCH, UCH)],
                                   ignored_value=-1)]
    return pltpu.async_copy(src, bufs[b].at[pl.ds(0, UCH)], usems[b])

  def _scatter_chunk(c, d):
    b = c & 1
    d.wait()
    base = c * UCH
    for v in range(UCH // LANES):
      wv = wtab[pl.ds(base + v * LANES, LANES)]
      row = iota + (lo + base + v * LANES)
      idx_stage[pl.ds(v * LANES, LANES)] = jnp.where(wv >= 0, row, -1)
    dst = out_hbm.at[plsc.Indices(idx_stage, ignored_value=-1)]
    pltpu.sync_copy(bufs[b].at[pl.ds(0, UCH)], dst)

  prev = _gather_chunk(0)
  for c in range(1, nup):
    cur = _gather_chunk(c)
    _scatter_chunk(c - 1, prev)
    prev = cur
  _scatter_chunk(nup - 1, prev)


def _sc_merge(node_ids, memory, new_h):
  f = pl.kernel(
      _merge_body,
      out_type=jax.ShapeDtypeStruct((NUM_NODES, MEM_DIM), jnp.float32),
      mesh=_mesh(),
      scratch_types=[
          pltpu.VMEM((BATCH,), jnp.int32),
          pltpu.VMEM((RPAD,), jnp.int32),
          pltpu.VMEM((UCH,), jnp.int32),
          pltpu.VMEM((CCH, MEM_DIM), jnp.float32),
          pltpu.VMEM((CCH, MEM_DIM), jnp.float32),
          pltpu.SemaphoreType.DMA,
          pltpu.SemaphoreType.DMA,
          pltpu.SemaphoreType.DMA,
          pltpu.SemaphoreType.DMA,
      ],
      compiler_params=pltpu.CompilerParams(needs_layout_passes=False),
  )
  return f(node_ids, memory, new_h)


def kernel(node_ids, messages, memory, W_ih, W_hh, b_ih, b_hh):
  node_ids = node_ids.astype(jnp.int32)
  b_ih2 = jnp.broadcast_to(b_ih.reshape(1, -1), (8, 3 * MEM_DIM))
  b_hh2 = jnp.broadcast_to(b_hh.reshape(1, -1), (8, 3 * MEM_DIM))
  h = _sc_gather(node_ids, memory)
  new_h = _tc_gru(messages, h, W_ih, W_hh, b_ih2, b_hh2)
  return _sc_merge(node_ids, memory, new_h)


# speedup vs baseline: 1.0724x; 1.0554x over previous
import jax
import jax.numpy as jnp
from jax import lax
from jax.experimental import pallas as pl
from jax.experimental.pallas import tpu as pltpu
from jax.experimental.pallas import tpu_sc as plsc

NUM_NODES = 100000
MEM_DIM = 128
BATCH = 16384

NC = 2
NS = 16
NW = NC * NS
LANES = 16

BPW = BATCH // NW
GCH = 128
RANGE = 3128
LAST_RANGE = NUM_NODES - (NW - 1) * RANGE
UCH = 128
CCH = 384
RPAD = ((RANGE + CCH - 1) // CCH) * CCH


def _widx():
  return lax.axis_index("s") * NC + lax.axis_index("c")


def _mesh():
  return plsc.VectorSubcoreMesh(core_axis_name="c", subcore_axis_name="s")


def _gather_scan_body(ids_hbm, mem_hbm, h_hbm, w_hbm,
                      ids2d, ids_all, wtab, rows_v, sem_g):
  wid = _widx()
  base = wid * BPW
  nch = BPW // GCH

  for k in range(nch):
    pltpu.sync_copy(ids_hbm.at[pl.ds(base + k * GCH, GCH)], ids2d.at[k])
  for k in range(nch):
    pltpu.async_copy(mem_hbm.at[ids2d.at[k]],
                     rows_v.at[pl.ds(k * GCH, GCH)], sem_g)

  lo = wid * RANGE
  pltpu.sync_copy(ids_hbm, ids_all)
  neg1 = jnp.full((LANES,), -1, jnp.int32)
  def _init(j, _):
    wtab[pl.ds(j * LANES, LANES)] = neg1
    return 0
  lax.fori_loop(0, RPAD // LANES, _init, 0, unroll=8)

  iota = lax.broadcasted_iota(jnp.int32, (LANES,), 0)
  def _scan(i, _):
    v = ids_all[pl.ds(i * LANES, LANES)]
    pos = iota + i * LANES
    _, win = plsc.scan_count(v)
    inr = jnp.logical_and(v >= lo, v < lo + RANGE)
    plsc.store_scatter(wtab, [v - lo], pos, mask=jnp.logical_and(win, inr))
    return 0
  lax.fori_loop(0, BATCH // LANES, _scan, 0, unroll=8)
  pltpu.sync_copy(wtab, w_hbm.at[pl.ds(wid * RPAD, RPAD)])

  for k in range(nch):
    pltpu.make_async_copy(mem_hbm.at[ids2d.at[k]],
                          rows_v.at[pl.ds(k * GCH, GCH)], sem_g).wait()
  pltpu.sync_copy(rows_v, h_hbm.at[pl.ds(base, BPW)])


def _sc_gather_scan(node_ids, memory):
  f = pl.kernel(
      _gather_scan_body,
      out_type=(
          jax.ShapeDtypeStruct((BATCH, MEM_DIM), jnp.float32),
          jax.ShapeDtypeStruct((NW * RPAD,), jnp.int32),
      ),
      mesh=_mesh(),
      scratch_types=[
          pltpu.VMEM((BPW // GCH, GCH), jnp.int32),
          pltpu.VMEM((BATCH,), jnp.int32),
          pltpu.VMEM((RPAD,), jnp.int32),
          pltpu.VMEM((BPW, MEM_DIM), jnp.float32),
          pltpu.SemaphoreType.DMA,
      ],
      compiler_params=pltpu.CompilerParams(needs_layout_passes=False),
  )
  return f(node_ids, memory)


def _gru_body(m_ref, h_ref, wih_ref, whh_ref, bih_ref, bhh_ref, out_ref):
  m = m_ref[...]
  h = h_ref[...]
  dn = (((1,), (1,)), ((), ()))
  gi = lax.dot_general(m, wih_ref[...], dn,
                       preferred_element_type=jnp.float32)
  gh = lax.dot_general(h, whh_ref[...], dn,
                       preferred_element_type=jnp.float32)
  gi = gi + bih_ref[0:1, :]
  gh = gh + bhh_ref[0:1, :]
  i_r = gi[:, 0:MEM_DIM]
  i_z = gi[:, MEM_DIM:2 * MEM_DIM]
  i_n = gi[:, 2 * MEM_DIM:3 * MEM_DIM]
  h_r = gh[:, 0:MEM_DIM]
  h_z = gh[:, MEM_DIM:2 * MEM_DIM]
  h_n = gh[:, 2 * MEM_DIM:3 * MEM_DIM]
  r = jax.nn.sigmoid(i_r + h_r)
  z = jax.nn.sigmoid(i_z + h_z)
  n = jnp.tanh(i_n + r * h_n)
  out_ref[...] = (1.0 - z) * n + z * h


def _tc_gru(messages, h, W_ih, W_hh, b_ih, b_hh):
  bm = 2048
  grid = (BATCH // bm,)
  return pl.pallas_call(
      _gru_body,
      grid=grid,
      in_specs=[
          pl.BlockSpec((bm, MEM_DIM), lambda i: (i, 0)),
          pl.BlockSpec((bm, MEM_DIM), lambda i: (i, 0)),
          pl.BlockSpec((3 * MEM_DIM, MEM_DIM), lambda i: (0, 0)),
          pl.BlockSpec((3 * MEM_DIM, MEM_DIM), lambda i: (0, 0)),
          pl.BlockSpec((8, 3 * MEM_DIM), lambda i: (0, 0)),
          pl.BlockSpec((8, 3 * MEM_DIM), lambda i: (0, 0)),
      ],
      out_specs=pl.BlockSpec((bm, MEM_DIM), lambda i: (i, 0)),
      out_shape=jax.ShapeDtypeStruct((BATCH, MEM_DIM), jnp.float32),
      compiler_params=pltpu.CompilerParams(
          dimension_semantics=("arbitrary",)),
  )(messages, h, W_ih, W_hh, b_ih, b_hh)


def _merge_body(mem_hbm, w_hbm, newh_hbm, out_hbm,
                wtab, cbuf0, cbuf1, sem_c0, sem_c1, sem_o0, sem_o1):
  wid = _widx()
  lo = wid * RANGE
  is_last = wid == NW - 1
  pltpu.sync_copy(w_hbm.at[pl.ds(wid * RPAD, RPAD)], wtab)

  bufs = (cbuf0, cbuf1)
  csems = (sem_c0, sem_c1)
  osems = (sem_o0, sem_o1)

  def _run(n_rows):
    sizes = [CCH] * (n_rows // CCH)
    if n_rows % CCH:
      sizes.append(n_rows % CCH)
    offs = [sum(sizes[:k]) for k in range(len(sizes))]

    def _issue_in(k):
      b = k & 1
      return pltpu.async_copy(mem_hbm.at[pl.ds(lo + offs[k], sizes[k])],
                              bufs[b].at[pl.ds(0, sizes[k])], csems[b])

    d_in = _issue_in(0)
    for k, (off, sz) in enumerate(zip(offs, sizes)):
      b = k & 1
      d_in.wait()
      ovs = []
      for so in range(0, sz, UCH):
        ss = min(UCH, sz - so)
        src = newh_hbm.at[plsc.Indices(wtab.at[pl.ds(off + so, ss)],
                                       ignored_value=-1)]
        ovs.append(pltpu.async_copy(src, bufs[b].at[pl.ds(so, ss)],
                                    osems[b]))
      if k + 1 < len(sizes):
        d_in = _issue_in(k + 1)
      for d in ovs:
        d.wait()
      pltpu.sync_copy(bufs[b].at[pl.ds(0, sz)],
                      out_hbm.at[pl.ds(lo + off, sz)])

  @pl.when(jnp.logical_not(is_last))
  def _():
    _run(RANGE)
  @pl.when(is_last)
  def _():
    _run(LAST_RANGE)


def _sc_merge(memory, wvec, new_h):
  f = pl.kernel(
      _merge_body,
      out_type=jax.ShapeDtypeStruct((NUM_NODES, MEM_DIM), jnp.float32),
      mesh=_mesh(),
      scratch_types=[
          pltpu.VMEM((RPAD,), jnp.int32),
          pltpu.VMEM((CCH, MEM_DIM), jnp.float32),
          pltpu.VMEM((CCH, MEM_DIM), jnp.float32),
          pltpu.SemaphoreType.DMA,
          pltpu.SemaphoreType.DMA,
          pltpu.SemaphoreType.DMA,
          pltpu.SemaphoreType.DMA,
      ],
      compiler_params=pltpu.CompilerParams(needs_layout_passes=False),
  )
  return f(memory, wvec, new_h)


def kernel(node_ids, messages, memory, W_ih, W_hh, b_ih, b_hh):
  node_ids = node_ids.astype(jnp.int32)
  b_ih2 = jnp.broadcast_to(b_ih.reshape(1, -1), (8, 3 * MEM_DIM))
  b_hh2 = jnp.broadcast_to(b_hh.reshape(1, -1), (8, 3 * MEM_DIM))
  h, wvec = _sc_gather_scan(node_ids, memory)
  new_h = _tc_gru(messages, h, W_ih, W_hh, b_ih2, b_hh2)
  return _sc_merge(memory, wvec, new_h)
